# baseline (speedup 1.0000x reference)
import math

import jax
import jax.numpy as jnp
from jax.experimental import pallas as pl

STRIDE = 16
SIZES = (128.0, 256.0, 512.0)
RATIOS = (0.5, 1.0, 2.0)
PRE_NMS = 6000
POST_NMS = 1000
NMS_THRESH = 0.7
MIN_SIZE = 1.0
BBOX_CLIP = math.log(1000.0 / 16.0)

H = W = 50
WP = 52
NCOL = H * WP


def _head_body(f_ref, w_ref, bc_ref, hw_ref, hb_ref, out_ref):
    patches = jnp.concatenate(
        [f_ref[:, (dy * WP + dx):(dy * WP + dx) + NCOL]
         for dy in range(3) for dx in range(3)], axis=0)
    t = jax.lax.dot_general(w_ref[...], patches, (((1,), (0,)), ((), ())),
                            preferred_element_type=jnp.float32)
    t = jnp.maximum(t + bc_ref[...], 0.0)
    out_ref[...] = jax.lax.dot_general(hw_ref[...], t, (((1,), (0,)), ((), ())),
                                       preferred_element_type=jnp.float32) + hb_ref[...]


def _rpn_head(feature, Wc, bc, Wo, bo, Wb, bb):
    f = feature[0]
    fpad = jnp.pad(f, ((0, 0), (1, 1), (1, 1))).reshape(256, WP * WP)
    fpad = jnp.pad(fpad, ((0, 0), (0, 8)))
    taps = jnp.transpose(Wc, (0, 2, 3, 1)).reshape(256, 9 * 256)
    hw = jnp.concatenate([Wo[:, :, 0, 0], Wb[:, :, 0, 0],
                          jnp.zeros((3, 256), jnp.float32)], axis=0)
    hb = jnp.concatenate([bo, bb, jnp.zeros((3,), jnp.float32)])[:, None]
    out = pl.pallas_call(
        _head_body,
        out_shape=jax.ShapeDtypeStruct((48, NCOL), jnp.float32),
    )(fpad, taps, bc[:, None], hw, hb)
    return out[0:9], out[9:45]


def _obj_scores(obj):
    o = obj.reshape(9, H, WP)[:, :, :W]
    return jnp.transpose(o, (1, 2, 0)).reshape(-1)


def _deltas(delta):
    d = delta.reshape(9, 4, H, WP)[:, :, :, :W]
    return jnp.transpose(d, (2, 3, 0, 1)).reshape(-1, 4)


def _make_anchors():
    sizes = jnp.asarray(SIZES, jnp.float32)
    ratios = jnp.asarray(RATIOS, jnp.float32)
    hr = jnp.sqrt(ratios)
    wr = 1.0 / hr
    ws = (wr[:, None] * sizes[None, :]).reshape(-1)
    hs = (hr[:, None] * sizes[None, :]).reshape(-1)
    base = jnp.stack([-ws / 2, -hs / 2, ws / 2, hs / 2], axis=1)
    sx = jnp.arange(W, dtype=jnp.float32) * STRIDE
    sy = jnp.arange(H, dtype=jnp.float32) * STRIDE
    yy, xx = jnp.meshgrid(sy, sx, indexing='ij')
    shifts = jnp.stack([xx, yy, xx, yy], axis=-1).reshape(-1, 4)
    return (shifts[:, None, :] + base[None, :, :]).reshape(-1, 4)


def _decode(delta, anchor):
    wa = anchor[:, 2] - anchor[:, 0]
    ha = anchor[:, 3] - anchor[:, 1]
    cxa = anchor[:, 0] + 0.5 * wa
    cya = anchor[:, 1] + 0.5 * ha
    dx = delta[:, 0] / 1.0
    dy = delta[:, 1] / 1.0
    dw = jnp.minimum(delta[:, 2] / 1.0, BBOX_CLIP)
    dh = jnp.minimum(delta[:, 3] / 1.0, BBOX_CLIP)
    cx = dx * wa + cxa
    cy = dy * ha + cya
    w = jnp.exp(dw) * wa
    h = jnp.exp(dh) * ha
    return jnp.stack([cx - 0.5 * w, cy - 0.5 * h, cx + 0.5 * w, cy + 0.5 * h], axis=1)


def _iou_one(box, boxes):
    lt = jnp.maximum(box[:2], boxes[:, :2])
    rb = jnp.minimum(box[2:], boxes[:, 2:])
    wh = jnp.clip(rb - lt, 0.0, None)
    inter = wh[:, 0] * wh[:, 1]
    a1 = (box[2] - box[0]) * (box[3] - box[1])
    a2 = (boxes[:, 2] - boxes[:, 0]) * (boxes[:, 3] - boxes[:, 1])
    return inter / (a1 + a2 - inter + 1e-9)


def kernel(feature, image_shape, Wc, bc, Wo, bo, Wb, bb):
    obj, delta = _rpn_head(feature, Wc, bc, Wo, bo, Wb, bb)
    objectness = _obj_scores(obj)
    pred = _deltas(delta)
    anchor = _make_anchors()
    pre_n = min(objectness.shape[0], PRE_NMS)
    score, top_idx = jax.lax.top_k(objectness, pre_n)
    proposal = _decode(pred[top_idx], anchor[top_idx])
    img = jnp.asarray(image_shape, jnp.float32)
    x1 = jnp.clip(proposal[:, 0], 0.0, img[1])
    y1 = jnp.clip(proposal[:, 1], 0.0, img[0])
    x2 = jnp.clip(proposal[:, 2], 0.0, img[1])
    y2 = jnp.clip(proposal[:, 3], 0.0, img[0])
    proposal = jnp.stack([x1, y1, x2, y2], axis=1)
    ww = x2 - x1
    hh = y2 - y1
    valid = (ww >= MIN_SIZE) & (hh >= MIN_SIZE)
    score = jnp.where(valid, score, -jnp.inf)
    order = jnp.argsort(-score)
    b = proposal[order]
    s = score[order]
    n = b.shape[0]

    def body(i, keep):
        iou = _iou_one(b[i], b)
        sup = (iou > NMS_THRESH) & (jnp.arange(n) > i) & keep[i]
        return keep & (~sup)

    keep = jax.lax.fori_loop(0, n, body, s > -jnp.inf)
    sel = jnp.where(keep, s, -jnp.inf)
    _, kidx = jax.lax.top_k(sel, POST_NMS)
    return b[kidx]


# speedup vs baseline: 72.5408x; 72.5408x over previous
import math

import jax
import jax.numpy as jnp
from jax.experimental import pallas as pl

STRIDE = 16
SIZES = (128.0, 256.0, 512.0)
RATIOS = (0.5, 1.0, 2.0)
PRE_NMS = 6000
POST_NMS = 1000
NMS_THRESH = 0.7
MIN_SIZE = 1.0
BBOX_CLIP = math.log(1000.0 / 16.0)

H = W = 50
WP = 52
NCOL = H * WP
NCOLP = 2688


def _conv(x, w, b, pad):
    y = jax.lax.conv_general_dilated(x, w, (1, 1), pad,
                                     dimension_numbers=('NCHW', 'OIHW', 'NCHW'))
    return y + b[None, :, None, None]


def _rpn_head(feature, Wc, bc, Wo, bo, Wb, bb):
    t = jax.nn.relu(_conv(feature, Wc, bc, 'SAME'))
    obj = _conv(t, Wo, bo, 'VALID')
    delta = _conv(t, Wb, bb, 'VALID')
    objectness = jnp.transpose(obj, (0, 2, 3, 1)).reshape(-1)
    pred = jnp.transpose(delta, (0, 2, 3, 1)).reshape(-1, 4)
    return objectness, pred


def _make_anchors():
    sizes = jnp.asarray(SIZES, jnp.float32)
    ratios = jnp.asarray(RATIOS, jnp.float32)
    hr = jnp.sqrt(ratios)
    wr = 1.0 / hr
    ws = (wr[:, None] * sizes[None, :]).reshape(-1)
    hs = (hr[:, None] * sizes[None, :]).reshape(-1)
    base = jnp.stack([-ws / 2, -hs / 2, ws / 2, hs / 2], axis=1)
    sx = jnp.arange(W, dtype=jnp.float32) * STRIDE
    sy = jnp.arange(H, dtype=jnp.float32) * STRIDE
    yy, xx = jnp.meshgrid(sy, sx, indexing='ij')
    shifts = jnp.stack([xx, yy, xx, yy], axis=-1).reshape(-1, 4)
    return (shifts[:, None, :] + base[None, :, :]).reshape(-1, 4)


def _decode(delta, anchor):
    wa = anchor[:, 2] - anchor[:, 0]
    ha = anchor[:, 3] - anchor[:, 1]
    cxa = anchor[:, 0] + 0.5 * wa
    cya = anchor[:, 1] + 0.5 * ha
    dx = delta[:, 0] / 1.0
    dy = delta[:, 1] / 1.0
    dw = jnp.minimum(delta[:, 2] / 1.0, BBOX_CLIP)
    dh = jnp.minimum(delta[:, 3] / 1.0, BBOX_CLIP)
    cx = dx * wa + cxa
    cy = dy * ha + cya
    w = jnp.exp(dw) * wa
    h = jnp.exp(dh) * ha
    return jnp.stack([cx - 0.5 * w, cy - 0.5 * h, cx + 0.5 * w, cy + 0.5 * h], axis=1)


NB = 48
NPAD = NB * 128
NOUT = 1024


def _colize(row, ncols):
    i0 = jax.lax.broadcasted_iota(jnp.int32, (128, 128), 0)
    i1 = jax.lax.broadcasted_iota(jnp.int32, (128, 128), 1)
    d = jnp.where(i0 == i1, jnp.broadcast_to(row, (128, 128)), 0.0)
    return jax.lax.dot_general(d, jnp.ones((128, ncols), jnp.float32),
                               (((1,), (0,)), ((), ())), precision=jax.lax.Precision.HIGHEST,
                               preferred_element_type=jnp.float32)


def _nms_body(x1_ref, y1_ref, x2_ref, y2_ref, valid_ref, out_ref,
              keep_ref, vcur_ref):
    vcur_ref[...] = valid_ref[...]
    area = (x2_ref[...] - x1_ref[...]) * (y2_ref[...] - y1_ref[...])
    jlt = (jax.lax.broadcasted_iota(jnp.int32, (128, 128), 0)
           < jax.lax.broadcasted_iota(jnp.int32, (128, 128), 1))

    def outer(b, _):
        x1b = x1_ref[pl.ds(b, 1), :]
        y1b = y1_ref[pl.ds(b, 1), :]
        x2b = x2_ref[pl.ds(b, 1), :]
        y2b = y2_ref[pl.ds(b, 1), :]
        ab = (x2b - x1b) * (y2b - y1b)
        X1 = _colize(x1b, 128)
        Y1 = _colize(y1b, 128)
        X2 = _colize(x2b, 128)
        Y2 = _colize(y2b, 128)
        AB = _colize(ab, 128)

        def iou_vs(x1t, y1t, x2t, y2t, at):
            w = jnp.maximum(jnp.minimum(X2, x2t) - jnp.maximum(X1, x1t), 0.0)
            h = jnp.maximum(jnp.minimum(Y2, y2t) - jnp.maximum(Y1, y1t), 0.0)
            inter = w * h
            return inter / (AB + at - inter + 1e-9)

        M = jnp.where((iou_vs(x1b, y1b, x2b, y2b, ab) > NMS_THRESH) & jlt,
                      1.0, 0.0)
        vb = vcur_ref[pl.ds(b, 1), :]

        def fix_cond(c):
            return c[1] > 0.5

        def fix_body(c):
            k, _ = c
            cnt = jax.lax.dot_general(k, M, (((1,), (0,)), ((), ())), precision=jax.lax.Precision.HIGHEST,
                                      preferred_element_type=jnp.float32)
            kn = jnp.where((cnt == 0.0), vb, 0.0)
            return kn, jnp.max(jnp.abs(kn - k))

        kb, _ = jax.lax.while_loop(fix_cond, fix_body, (vb, jnp.float32(1.0)))
        keep_ref[pl.ds(b, 1), :] = kb

        def inner(t, _):
            x1t = x1_ref[pl.ds(t, 1), :]
            y1t = y1_ref[pl.ds(t, 1), :]
            x2t = x2_ref[pl.ds(t, 1), :]
            y2t = y2_ref[pl.ds(t, 1), :]
            at = (x2t - x1t) * (y2t - y1t)
            Mt = jnp.where(iou_vs(x1t, y1t, x2t, y2t, at) > NMS_THRESH,
                           1.0, 0.0)
            cnt = jax.lax.dot_general(kb, Mt, (((1,), (0,)), ((), ())), precision=jax.lax.Precision.HIGHEST,
                                      preferred_element_type=jnp.float32)
            vcur_ref[pl.ds(t, 1), :] = jnp.where(
                cnt == 0.0, vcur_ref[pl.ds(t, 1), :], 0.0)
            return 0

        jax.lax.fori_loop(b + 1, NB, inner, 0)
        return 0

    jax.lax.fori_loop(0, NB, outer, 0)

    keep = keep_ref[...]
    considered = (jax.lax.broadcasted_iota(jnp.int32, (NB, 128), 0) * 128
                  + jax.lax.broadcasted_iota(jnp.int32, (NB, 128), 1)) < PRE_NMS
    filler = jnp.where(considered & (keep < 0.5), 1.0, 0.0)
    U = jnp.where(jax.lax.broadcasted_iota(jnp.int32, (128, 128), 0)
                  <= jax.lax.broadcasted_iota(jnp.int32, (128, 128), 1),
                  1.0, 0.0)
    L48 = jnp.where(jax.lax.broadcasted_iota(jnp.int32, (NB, NB), 0)
                    > jax.lax.broadcasted_iota(jnp.int32, (NB, NB), 1),
                    1.0, 0.0)

    def rowmajor_cumsum(m):
        rc = jax.lax.dot_general(m, U, (((1,), (0,)), ((), ())), precision=jax.lax.Precision.HIGHEST,
                                 preferred_element_type=jnp.float32)
        tot = rc[:, 127:128]
        offs = jax.lax.dot_general(L48, tot, (((1,), (0,)), ((), ())), precision=jax.lax.Precision.HIGHEST,
                                   preferred_element_type=jnp.float32)
        return rc + offs, jnp.sum(tot)

    ck, ktot = rowmajor_cumsum(keep)
    cf, _ = rowmajor_cumsum(filler)
    vcur_ref[...] = jnp.where(keep > 0.5, ck - 1.0,
                              jnp.where(filler > 0.5, ktot + cf - 1.0,
                                        jnp.float32(NPAD + NOUT)))

    iota_p = jax.lax.broadcasted_iota(jnp.int32, (128, NOUT), 1).astype(jnp.float32)
    zero4 = jnp.zeros((4, 128), jnp.float32)

    def emit(r, acc):
        dr = vcur_ref[pl.ds(r, 1), :]
        C = _colize(dr, NOUT)
        oh = jnp.where(C == iota_p, 1.0, 0.0)
        V = jnp.concatenate([x1_ref[pl.ds(r, 1), :], y1_ref[pl.ds(r, 1), :],
                             x2_ref[pl.ds(r, 1), :], y2_ref[pl.ds(r, 1), :],
                             zero4], axis=0)
        return acc + jax.lax.dot_general(V, oh, (((1,), (0,)), ((), ())), precision=jax.lax.Precision.HIGHEST,
                                         preferred_element_type=jnp.float32)

    out_ref[...] = jax.lax.fori_loop(0, NB, emit,
                                     jnp.zeros((8, NOUT), jnp.float32))


def _nms_select(b, valid6000):
    def to48(v):
        return jnp.pad(v, (0, NPAD - PRE_NMS)).reshape(NB, 128)

    x1 = to48(b[:, 0])
    y1 = to48(b[:, 1])
    x2 = to48(b[:, 2])
    y2 = to48(b[:, 3])
    vv = to48(valid6000.astype(jnp.float32))
    from jax.experimental.pallas import tpu as pltpu
    out = pl.pallas_call(
        _nms_body,
        out_shape=jax.ShapeDtypeStruct((8, NOUT), jnp.float32),
        scratch_shapes=[pltpu.VMEM((NB, 128), jnp.float32),
                        pltpu.VMEM((NB, 128), jnp.float32)],
    )(x1, y1, x2, y2, vv)
    return out[:4, :POST_NMS].T


def kernel(feature, image_shape, Wc, bc, Wo, bo, Wb, bb):
    objectness, pred = _rpn_head(feature, Wc, bc, Wo, bo, Wb, bb)
    anchor = _make_anchors()
    pre_n = min(objectness.shape[0], PRE_NMS)
    score, top_idx = jax.lax.top_k(objectness, pre_n)
    proposal = _decode(pred[top_idx], anchor[top_idx])
    img = jnp.asarray(image_shape, jnp.float32)
    x1 = jnp.clip(proposal[:, 0], 0.0, img[1])
    y1 = jnp.clip(proposal[:, 1], 0.0, img[0])
    x2 = jnp.clip(proposal[:, 2], 0.0, img[1])
    y2 = jnp.clip(proposal[:, 3], 0.0, img[0])
    proposal = jnp.stack([x1, y1, x2, y2], axis=1)
    ww = x2 - x1
    hh = y2 - y1
    valid = (ww >= MIN_SIZE) & (hh >= MIN_SIZE)
    score = jnp.where(valid, score, -jnp.inf)
    order = jnp.argsort(-score)
    b = proposal[order]
    s = score[order]
    b, s = jax.lax.optimization_barrier((b, s))
    return _nms_select(b, s > -jnp.inf)
